# baseline (speedup 1.0000x reference)
import functools

import jax
import jax.numpy as jnp
import numpy as np
from jax import lax
from jax.experimental import pallas as pl
from jax.experimental.pallas import tpu as pltpu
from jax.experimental.pallas import tpu_sc as plsc

N = 10000
E = 160000
D_IN = 128
D_EDGE = 16
H = 10
HP = 16
C = 64
L = 2
G = 64
OUT = 128

NC = 2
NS = 16
NW = NC * NS
CH = 128
EPAD = 163840
CPW = EPAD // (NW * CH)
ROWS_PER_TILE = 624
ROWS_LAST = N - (NS - 1) * ROWS_PER_TILE

_f32 = jnp.float32
_bf16 = jnp.bfloat16
_XPERM = np.concatenate([
    32 * g + np.ravel(np.stack([np.arange(16), 16 + np.arange(16)], axis=1))
    for g in range(H * C // 32)
])
_mesh = plsc.VectorSubcoreMesh(core_axis_name="c", subcore_axis_name="s")
_sc_params = pltpu.CompilerParams(use_tc_tiling_on_sc=False,
                                  needs_layout_passes=False)
_tc_params = pltpu.CompilerParams(vmem_limit_bytes=64 * 1024 * 1024)


def _node0_body(x_ref, we_ref, be_ref, cw_ref, ws_ref, wd_ref,
                h_ref, xp_ref, aa_ref):
    h = jnp.dot(x_ref[...].astype(_bf16), we_ref[...],
                preferred_element_type=_f32)
    h = h + be_ref[...]
    h_ref[...] = h
    hb = h.astype(_bf16)
    xp_ref[...] = jnp.dot(hb, cw_ref[...],
                          preferred_element_type=_f32).astype(_bf16)
    aa_ref[0:N, :] = jnp.dot(hb, ws_ref[...], preferred_element_type=_f32)
    aa_ref[N:2 * N, :] = jnp.dot(hb, wd_ref[...], preferred_element_type=_f32)


def _edge_body(ea_ref, w0_ref, w1_ref, o0_ref, o1_ref):
    i = pl.program_id(0)
    ea = ea_ref[...]
    rows = i * _EB + lax.broadcasted_iota(jnp.int32, (_EB, HP), 0)
    pad = rows >= E
    o0 = jnp.dot(ea, w0_ref[...], preferred_element_type=_f32)
    o1 = jnp.dot(ea, w1_ref[...], preferred_element_type=_f32)
    o0_ref[...] = jnp.where(pad, -1e30, o0)
    o1_ref[...] = jnp.where(pad, -1e30, o1)


def _bn_gelu_res(o0, o1, bias, gamma, beta, hprev):
    out = o0 + o1 + bias
    mu = jnp.mean(out, axis=0, keepdims=True)
    var = jnp.mean((out - mu) ** 2, axis=0, keepdims=True)
    xn = (out - mu) / jnp.sqrt(var + 1e-5) * gamma + beta
    g = 0.5 * xn * (1.0 + lax.erf(xn / jnp.sqrt(2.0).astype(_f32)))
    return hprev + g


def _post_body(o0_ref, o1_ref, bias_ref, gam_ref, bet_ref, hp_ref,
               cw_ref, ws_ref, wd_ref,
               h_ref, xp_ref, aa_ref):
    h = _bn_gelu_res(o0_ref[...], o1_ref[...], bias_ref[...], gam_ref[...],
                     bet_ref[...], hp_ref[...])
    h_ref[...] = h
    hb = h.astype(_bf16)
    xp_ref[...] = jnp.dot(hb, cw_ref[...],
                          preferred_element_type=_f32).astype(_bf16)
    aa_ref[0:N, :] = jnp.dot(hb, ws_ref[...], preferred_element_type=_f32)
    aa_ref[N:2 * N, :] = jnp.dot(hb, wd_ref[...], preferred_element_type=_f32)


def _final_body(o0_ref, o1_ref, bias_ref, gam_ref, bet_ref, hp_ref,
                lw_ref, lb_ref, batch_ref, out_ref):
    h = _bn_gelu_res(o0_ref[...], o1_ref[...], bias_ref[...], gam_ref[...],
                     bet_ref[...], hp_ref[...])
    y = jnp.dot(h, lw_ref[...], preferred_element_type=_f32) + lb_ref[...]
    gids = lax.broadcasted_iota(jnp.int32, (G, N), 0)
    oh = (gids == batch_ref[...]).astype(_f32)
    ssum = jnp.dot(oh, y, preferred_element_type=_f32)
    cnt = jnp.sum(oh, axis=1, keepdims=True)
    out_ref[...] = ssum / jnp.maximum(cnt, 1.0)


_node0_call = pl.pallas_call(
    _node0_body,
    compiler_params=_tc_params,
    out_shape=[
        jax.ShapeDtypeStruct((N, C), _f32),
        jax.ShapeDtypeStruct((N, H * C), _bf16),
        jax.ShapeDtypeStruct((2 * N, HP), _f32),
    ],
)

_EB = 8192
_edge_call = pl.pallas_call(
    _edge_body,
    grid=(EPAD // _EB,),
    in_specs=[
        pl.BlockSpec((_EB, D_EDGE), lambda i: (i, 0)),
        pl.BlockSpec((D_EDGE, HP), lambda i: (0, 0)),
        pl.BlockSpec((D_EDGE, HP), lambda i: (0, 0)),
    ],
    out_specs=[
        pl.BlockSpec((_EB, HP), lambda i: (i, 0)),
        pl.BlockSpec((_EB, HP), lambda i: (i, 0)),
    ],
    out_shape=[
        jax.ShapeDtypeStruct((EPAD, HP), _f32),
        jax.ShapeDtypeStruct((EPAD, HP), _f32),
    ],
)

_post_call = pl.pallas_call(
    _post_body,
    compiler_params=_tc_params,
    out_shape=[
        jax.ShapeDtypeStruct((N, C), _f32),
        jax.ShapeDtypeStruct((N, H * C), _bf16),
        jax.ShapeDtypeStruct((2 * N, HP), _f32),
    ],
)

_final_call = pl.pallas_call(
    _final_body,
    compiler_params=_tc_params,
    out_shape=jax.ShapeDtypeStruct((G, OUT), _f32),
)


def _dump_acc(acc, out_hbm, s):
    @pl.when(s < NS - 1)
    def _():
        pltpu.sync_copy(acc.at[pl.ds(s * ROWS_PER_TILE, ROWS_PER_TILE)],
                        out_hbm.at[pl.ds(s * ROWS_PER_TILE, ROWS_PER_TILE)])

    @pl.when(s == NS - 1)
    def _():
        pltpu.sync_copy(acc.at[pl.ds((NS - 1) * ROWS_PER_TILE, ROWS_LAST)],
                        out_hbm.at[pl.ds((NS - 1) * ROWS_PER_TILE, ROWS_LAST)])


@functools.partial(
    pl.kernel,
    mesh=_mesh,
    compiler_params=_sc_params,
    out_type=[
        jax.ShapeDtypeStruct((EPAD, HP), _f32),
        jax.ShapeDtypeStruct((NC, N, HP), _f32),
    ],
    scratch_types=[
        pltpu.VMEM((CPW, CH), jnp.int32),
        pltpu.VMEM((CPW, CH), jnp.int32),
        pltpu.VMEM((CPW, CH), jnp.int32),
        pltpu.VMEM((CH, HP), _f32),
        pltpu.VMEM((CH, HP), _f32),
        pltpu.VMEM((CH, HP), _f32),
        pltpu.VMEM((CH, HP), _f32),
        pltpu.VMEM((CH, HP), _f32),
        pltpu.VMEM((CH, HP), _f32),
        pltpu.VMEM((CH, HP), _f32),
        pltpu.VMEM((CH, HP), _f32),
        pltpu.VMEM_SHARED((N, HP), _f32),
        pltpu.SemaphoreType.DMA,
        pltpu.SemaphoreType.DMA,
        pltpu.SemaphoreType.DMA,
        pltpu.SemaphoreType.DMA,
        pltpu.SemaphoreType.DMA,
        pltpu.SemaphoreType.DMA,
        pltpu.SemaphoreType.DMA,
        pltpu.SemaphoreType.DMA,
        pltpu.SemaphoreType.DMA,
        pltpu.SemaphoreType.DMA,
    ],
)
def _sc_attn(srcv_hbm, dstnv_hbm, dstv_hbm, aa_hbm, aedge_hbm, zero_hbm,
             ex_hbm, dpart_hbm,
             sidx, nidx, didx, asg0, asg1, adg0, adg1, ae0, ae1, ex0, ex1,
             dacc, sa0, sa1, sb0, sb1, sc0, sc1, sw0, sw1, ss0, ss1):
    c = lax.axis_index("c")
    s = lax.axis_index("s")
    wid = c * NS + s
    asg = (asg0, asg1)
    adg = (adg0, adg1)
    ae = (ae0, ae1)
    ex = (ex0, ex1)
    sa = (sa0, sa1)
    sb = (sb0, sb1)
    se = (sc0, sc1)
    sw = (sw0, sw1)
    ss = (ss0, ss1)

    @pl.when(s == 0)
    def _():
        pltpu.sync_copy(zero_hbm, dacc)

    pltpu.sync_copy(srcv_hbm.at[pl.ds(wid * CPW, CPW)], sidx)
    pltpu.sync_copy(dstnv_hbm.at[pl.ds(wid * CPW, CPW)], nidx)
    pltpu.sync_copy(dstv_hbm.at[pl.ds(wid * CPW, CPW)], didx)
    plsc.subcore_barrier()

    def start(i, b):
        pltpu.async_copy(aa_hbm.at[sidx.at[i]], asg[b], sa[b])
        pltpu.async_copy(aa_hbm.at[nidx.at[i]], adg[b], sb[b])
        pltpu.async_copy(aedge_hbm.at[pl.ds((wid * CPW + i) * CH, CH)],
                         ae[b], se[b])

    def wait(b):
        pltpu.make_async_copy(aa_hbm.at[sidx.at[0]], asg[b], sa[b]).wait()
        pltpu.make_async_copy(aa_hbm.at[nidx.at[0]], adg[b], sb[b]).wait()
        pltpu.make_async_copy(aedge_hbm.at[pl.ds(0, CH)], ae[b], se[b]).wait()

    def wait_out(b):
        pltpu.make_async_copy(ex[b], ex_hbm.at[pl.ds(0, CH)], sw[b]).wait()
        pltpu.make_async_copy(ex[b], dacc.at[didx.at[0]], ss[b]).wait()

    def step(i, b, b2):
        wait(b)

        @pl.when(i + 1 < CPW)
        def _():
            start(i + 1, b2)

        @pl.when(i >= 2)
        def _():
            wait_out(b)

        @pl.loop(0, CH, step=4)
        def _(k):
            for j in range(4):
                a = asg[b][k + j, :] + adg[b][k + j, :] + ae[b][k + j, :]
                a = jnp.maximum(a, 0.2 * a)
                ex[b][k + j, :] = jnp.exp(a)

        pltpu.async_copy(ex[b], ex_hbm.at[pl.ds((wid * CPW + i) * CH, CH)],
                         sw[b])
        pltpu.async_copy(ex[b], dacc.at[didx.at[i]], ss[b], add=True)

    start(0, 0)

    @pl.loop(0, CPW)
    def _(i):
        even = (i % 2) == 0

        @pl.when(even)
        def _():
            step(i, 0, 1)

        @pl.when(jnp.logical_not(even))
        def _():
            step(i, 1, 0)

    wait_out(0)
    wait_out(1)
    plsc.subcore_barrier()
    _dump_acc(dacc, dpart_hbm.at[c], s)


@functools.partial(
    pl.kernel,
    mesh=_mesh,
    compiler_params=_sc_params,
    out_type=jax.ShapeDtypeStruct((NC, N, C), _f32),
    scratch_types=[
        pltpu.VMEM((CPW, CH), jnp.int32),
        pltpu.VMEM((CPW, CH), jnp.int32),
        pltpu.VMEM((CH // 2, H * C), _bf16),
        pltpu.VMEM((CH // 2, H * C), _bf16),
        pltpu.VMEM((CH, HP), _f32),
        pltpu.VMEM((CH, HP), _f32),
        pltpu.VMEM((CH, HP), _f32),
        pltpu.VMEM((CH, HP), _f32),
        pltpu.VMEM((CH, HP), _f32),
        pltpu.VMEM((CH, HP), _f32),
        pltpu.VMEM((CH, C), _f32),
        pltpu.VMEM((CH, C), _f32),
        pltpu.VMEM_SHARED((N, C), _f32),
        pltpu.SemaphoreType.DMA,
        pltpu.SemaphoreType.DMA,
        pltpu.SemaphoreType.DMA,
        pltpu.SemaphoreType.DMA,
        pltpu.SemaphoreType.DMA,
        pltpu.SemaphoreType.DMA,
        pltpu.SemaphoreType.DMA,
        pltpu.SemaphoreType.DMA,
        pltpu.SemaphoreType.DMA,
        pltpu.SemaphoreType.DMA,
    ],
)
def _sc_msg(srcv_hbm, dstv_hbm, xp_hbm, ex_hbm, d0_hbm, d1_hbm, zero_hbm,
            outp_hbm,
            sidx, didx, xp0, xp1, exb0, exb1, d00, d01, d10, d11,
            msg0, msg1, oacc,
            sx0, sx1, sE0, sE1, s00, s01, s10, s11, sm0, sm1):
    c = lax.axis_index("c")
    s = lax.axis_index("s")
    wid = c * NS + s
    HB = CH // 2
    xp = (xp0, xp1)
    exb = (exb0, exb1)
    d0 = (d00, d01)
    d1 = (d10, d11)
    msg = (msg0, msg1)
    sx = (sx0, sx1)
    sE = (sE0, sE1)
    s0 = (s00, s01)
    s1 = (s10, s11)
    sm = (sm0, sm1)

    @pl.when(s == 0)
    def _():
        pltpu.sync_copy(zero_hbm, oacc)

    pltpu.sync_copy(srcv_hbm.at[pl.ds(wid * CPW, CPW)], sidx)
    pltpu.sync_copy(dstv_hbm.at[pl.ds(wid * CPW, CPW)], didx)
    plsc.subcore_barrier()

    def start_small(i, b):
        pltpu.async_copy(d0_hbm.at[didx.at[i]], d0[b], s0[b])
        pltpu.async_copy(d1_hbm.at[didx.at[i]], d1[b], s1[b])
        pltpu.async_copy(ex_hbm.at[pl.ds((wid * CPW + i) * CH, CH)],
                         exb[b], sE[b])

    def wait_small(b):
        pltpu.make_async_copy(d0_hbm.at[didx.at[0]], d0[b], s0[b]).wait()
        pltpu.make_async_copy(d1_hbm.at[didx.at[0]], d1[b], s1[b]).wait()
        pltpu.make_async_copy(ex_hbm.at[pl.ds(0, CH)], exb[b], sE[b]).wait()

    def start_xp(i, hf, xb):
        pltpu.async_copy(xp_hbm.at[sidx.at[i, pl.ds(hf * HB, HB)]],
                         xp[xb], sx[xb])

    def wait_xp(xb):
        pltpu.make_async_copy(xp_hbm.at[sidx.at[0, pl.ds(0, HB)]],
                              xp[xb], sx[xb]).wait()

    def compute_half(b, hf, xb):
        base = hf * HB

        @pl.loop(0, HB, step=2)
        def _(k0):
            for j in range(2):
                k = base + k0 + j
                den = d0[b][k, :] + d1[b][k, :] + 1e-16
                wrow = exb[b][k, :] / den * (1.0 / H)
                accs = [jnp.zeros((16,), _f32) for _ in range(4)]
                for hh in range(H):
                    wk = wrow[hh]
                    for half in range(2):
                        v = xp[xb][k0 + j, pl.ds(hh * C + half * 32, 32)]
                        lo, hi = plsc.unpack(
                            v, format=plsc.PackFormat.INTERLEAVED)
                        accs[2 * half] = accs[2 * half] + wk * lo
                        accs[2 * half + 1] = accs[2 * half + 1] + wk * hi
                for q in range(4):
                    msg[b][k, pl.ds(q * 16, 16)] = accs[q]

    def wait_out(b):
        pltpu.make_async_copy(msg[b], oacc.at[didx.at[0]], sm[b]).wait()

    def step(i, b, b2):
        wait_small(b)

        @pl.when(i + 1 < CPW)
        def _():
            start_small(i + 1, b2)

        @pl.when(i >= 2)
        def _():
            wait_out(b)

        wait_xp(0)
        start_xp(i, 1, 1)
        compute_half(b, 0, 0)
        wait_xp(1)

        @pl.when(i + 1 < CPW)
        def _():
            start_xp(i + 1, 0, 0)

        compute_half(b, 1, 1)
        pltpu.async_copy(msg[b], oacc.at[didx.at[i]], sm[b], add=True)

    start_small(0, 0)
    start_xp(0, 0, 0)

    @pl.loop(0, CPW)
    def _(i):
        even = (i % 2) == 0

        @pl.when(even)
        def _():
            step(i, 0, 1)

        @pl.when(jnp.logical_not(even))
        def _():
            step(i, 1, 0)

    wait_out(0)
    wait_out(1)
    plsc.subcore_barrier()
    _dump_acc(oacc, outp_hbm.at[c], s)


def kernel(x, edge_index, edge_attr, batch, W_embed, b_embed, convW, edgeW,
           att_src, att_dst, att_edge, conv_bias, bn_gamma, bn_beta,
           linW, linb):
    padi = jnp.zeros((EPAD - E,), jnp.int32)
    src1 = jnp.concatenate([edge_index[0].astype(jnp.int32), padi])
    dst1 = jnp.concatenate([edge_index[1].astype(jnp.int32), padi])
    srcv = src1.reshape(EPAD // CH, CH)
    dstv = dst1.reshape(EPAD // CH, CH)
    dstnv = dstv + N

    cw = convW.reshape(L, C, H, C)
    ws = jnp.einsum("lchd,lhd->lch", cw, att_src)
    wd = jnp.einsum("lchd,lhd->lch", cw, att_dst)
    ew = edgeW.reshape(L, D_EDGE, H, C)
    we = jnp.einsum("lehd,lhd->leh", ew, att_edge)
    pad = ((0, 0), (0, 0), (0, HP - H))
    ws = jnp.pad(ws, pad).astype(_bf16)
    wd = jnp.pad(wd, pad).astype(_bf16)
    we = jnp.pad(we, pad)
    cwp = convW[:, :, _XPERM].astype(_bf16)
    web = W_embed.astype(_bf16)

    zeros16 = jnp.zeros((N, HP), _f32)
    zeros64 = jnp.zeros((N, C), _f32)
    ea_pad = jnp.concatenate(
        [edge_attr, jnp.zeros((EPAD - E, D_EDGE), _f32)])

    aedge0, aedge1 = _edge_call(ea_pad, we[0], we[1])

    h0, xp0, aa0 = _node0_call(x, web, b_embed.reshape(1, C),
                               cwp[0], ws[0], wd[0])

    ex0, dp0 = _sc_attn(srcv, dstnv, dstv, aa0, aedge0, zeros16)
    op0 = _sc_msg(srcv, dstv, xp0, ex0, dp0[0], dp0[1], zeros64)

    h1, xp1, aa1 = _post_call(op0[0], op0[1],
                              conv_bias[0].reshape(1, C),
                              bn_gamma[0].reshape(1, C),
                              bn_beta[0].reshape(1, C),
                              h0, cwp[1], ws[1], wd[1])

    ex1, dp1 = _sc_attn(srcv, dstnv, dstv, aa1, aedge1, zeros16)
    op1 = _sc_msg(srcv, dstv, xp1, ex1, dp1[0], dp1[1], zeros64)

    pooled = _final_call(op1[0], op1[1],
                         conv_bias[1].reshape(1, C),
                         bn_gamma[1].reshape(1, C),
                         bn_beta[1].reshape(1, C),
                         h1, linW, linb.reshape(1, OUT),
                         batch.reshape(1, N).astype(jnp.int32))
    return pooled


# speedup vs baseline: 1.4050x; 1.4050x over previous
import functools

import jax
import jax.numpy as jnp
import numpy as np
from jax import lax
from jax.experimental import pallas as pl
from jax.experimental.pallas import tpu as pltpu
from jax.experimental.pallas import tpu_sc as plsc

N = 10000
E = 160000
D_IN = 128
D_EDGE = 16
H = 10
HP = 16
C = 64
L = 2
G = 64
OUT = 128

NC = 2
NS = 16
NW = NC * NS
CH = 128
EPAD = 163840
CPW = EPAD // (NW * CH)
ROWS_PER_TILE = 624
ROWS_LAST = N - (NS - 1) * ROWS_PER_TILE

_f32 = jnp.float32
_bf16 = jnp.bfloat16
_XPERM = np.concatenate([
    32 * g + np.ravel(np.stack([np.arange(16), 16 + np.arange(16)], axis=1))
    for g in range(H * C // 32)
])
_mesh = plsc.VectorSubcoreMesh(core_axis_name="c", subcore_axis_name="s")
_sc_params = pltpu.CompilerParams(use_tc_tiling_on_sc=False,
                                  needs_layout_passes=False)
_tc_params = pltpu.CompilerParams(vmem_limit_bytes=64 * 1024 * 1024)


def _node0_body(x_ref, we_ref, be_ref, cw_ref, ws_ref, wd_ref,
                h_ref, xp_ref, aa_ref):
    h = jnp.dot(x_ref[...].astype(_bf16), we_ref[...],
                preferred_element_type=_f32)
    h = h + be_ref[...]
    h_ref[...] = h
    hb = h.astype(_bf16)
    xp_ref[...] = jnp.dot(hb, cw_ref[...],
                          preferred_element_type=_f32).astype(_bf16)
    aa_ref[0:N, :] = jnp.dot(hb, ws_ref[...], preferred_element_type=_f32)
    aa_ref[N:2 * N, :] = jnp.dot(hb, wd_ref[...], preferred_element_type=_f32)


def _edge_body(ea_ref, w0_ref, w1_ref, o0_ref, o1_ref):
    i = pl.program_id(0)
    ea = ea_ref[...]
    rows = i * _EB + lax.broadcasted_iota(jnp.int32, (_EB, HP), 0)
    pad = rows >= E
    o0 = jnp.dot(ea, w0_ref[...], preferred_element_type=_f32)
    o1 = jnp.dot(ea, w1_ref[...], preferred_element_type=_f32)
    o0_ref[...] = jnp.where(pad, -1e30, o0)
    o1_ref[...] = jnp.where(pad, -1e30, o1)


def _bn_gelu_res(o0, o1, bias, gamma, beta, hprev):
    out = o0 + o1 + bias
    mu = jnp.mean(out, axis=0, keepdims=True)
    var = jnp.mean((out - mu) ** 2, axis=0, keepdims=True)
    xn = (out - mu) / jnp.sqrt(var + 1e-5) * gamma + beta
    g = 0.5 * xn * (1.0 + lax.erf(xn / jnp.sqrt(2.0).astype(_f32)))
    return hprev + g


def _post_body(o0_ref, o1_ref, bias_ref, gam_ref, bet_ref, hp_ref,
               cw_ref, ws_ref, wd_ref,
               h_ref, xp_ref, aa_ref):
    h = _bn_gelu_res(o0_ref[...], o1_ref[...], bias_ref[...], gam_ref[...],
                     bet_ref[...], hp_ref[...])
    h_ref[...] = h
    hb = h.astype(_bf16)
    xp_ref[...] = jnp.dot(hb, cw_ref[...],
                          preferred_element_type=_f32).astype(_bf16)
    aa_ref[0:N, :] = jnp.dot(hb, ws_ref[...], preferred_element_type=_f32)
    aa_ref[N:2 * N, :] = jnp.dot(hb, wd_ref[...], preferred_element_type=_f32)


def _final_body(o0_ref, o1_ref, bias_ref, gam_ref, bet_ref, hp_ref,
                lw_ref, lb_ref, batch_ref, out_ref):
    h = _bn_gelu_res(o0_ref[...], o1_ref[...], bias_ref[...], gam_ref[...],
                     bet_ref[...], hp_ref[...])
    y = jnp.dot(h, lw_ref[...], preferred_element_type=_f32) + lb_ref[...]
    gids = lax.broadcasted_iota(jnp.int32, (G, N), 0)
    oh = (gids == batch_ref[...]).astype(_f32)
    ssum = jnp.dot(oh, y, preferred_element_type=_f32)
    cnt = jnp.sum(oh, axis=1, keepdims=True)
    out_ref[...] = ssum / jnp.maximum(cnt, 1.0)


_node0_call = pl.pallas_call(
    _node0_body,
    compiler_params=_tc_params,
    out_shape=[
        jax.ShapeDtypeStruct((N, C), _f32),
        jax.ShapeDtypeStruct((N, H * C), _bf16),
        jax.ShapeDtypeStruct((2 * N, HP), _f32),
    ],
)

_EB = 8192
_edge_call = pl.pallas_call(
    _edge_body,
    grid=(EPAD // _EB,),
    in_specs=[
        pl.BlockSpec((_EB, D_EDGE), lambda i: (i, 0)),
        pl.BlockSpec((D_EDGE, HP), lambda i: (0, 0)),
        pl.BlockSpec((D_EDGE, HP), lambda i: (0, 0)),
    ],
    out_specs=[
        pl.BlockSpec((_EB, HP), lambda i: (i, 0)),
        pl.BlockSpec((_EB, HP), lambda i: (i, 0)),
    ],
    out_shape=[
        jax.ShapeDtypeStruct((EPAD, HP), _f32),
        jax.ShapeDtypeStruct((EPAD, HP), _f32),
    ],
)

_post_call = pl.pallas_call(
    _post_body,
    compiler_params=_tc_params,
    out_shape=[
        jax.ShapeDtypeStruct((N, C), _f32),
        jax.ShapeDtypeStruct((N, H * C), _bf16),
        jax.ShapeDtypeStruct((2 * N, HP), _f32),
    ],
)

_final_call = pl.pallas_call(
    _final_body,
    compiler_params=_tc_params,
    out_shape=jax.ShapeDtypeStruct((G, OUT), _f32),
)


def _dump_acc(acc, out_hbm, s):
    @pl.when(s < NS - 1)
    def _():
        pltpu.sync_copy(acc.at[pl.ds(s * ROWS_PER_TILE, ROWS_PER_TILE)],
                        out_hbm.at[pl.ds(s * ROWS_PER_TILE, ROWS_PER_TILE)])

    @pl.when(s == NS - 1)
    def _():
        pltpu.sync_copy(acc.at[pl.ds((NS - 1) * ROWS_PER_TILE, ROWS_LAST)],
                        out_hbm.at[pl.ds((NS - 1) * ROWS_PER_TILE, ROWS_LAST)])


@functools.partial(
    pl.kernel,
    mesh=_mesh,
    compiler_params=_sc_params,
    out_type=[
        jax.ShapeDtypeStruct((EPAD, HP), _f32),
        jax.ShapeDtypeStruct((NC, N, HP), _f32),
    ],
    scratch_types=[
        pltpu.VMEM((CPW, CH), jnp.int32),
        pltpu.VMEM((CPW, CH), jnp.int32),
        pltpu.VMEM((CPW, CH), jnp.int32),
        pltpu.VMEM((CH, HP), _f32),
        pltpu.VMEM((CH, HP), _f32),
        pltpu.VMEM((CH, HP), _f32),
        pltpu.VMEM((CH, HP), _f32),
        pltpu.VMEM((CH, HP), _f32),
        pltpu.VMEM((CH, HP), _f32),
        pltpu.VMEM((CH, HP), _f32),
        pltpu.VMEM((CH, HP), _f32),
        pltpu.VMEM_SHARED((N, HP), _f32),
        pltpu.SemaphoreType.DMA,
        pltpu.SemaphoreType.DMA,
        pltpu.SemaphoreType.DMA,
        pltpu.SemaphoreType.DMA,
        pltpu.SemaphoreType.DMA,
        pltpu.SemaphoreType.DMA,
        pltpu.SemaphoreType.DMA,
        pltpu.SemaphoreType.DMA,
        pltpu.SemaphoreType.DMA,
        pltpu.SemaphoreType.DMA,
    ],
)
def _sc_attn(srcv_hbm, dstnv_hbm, dstv_hbm, aa_hbm, aedge_hbm, zero_hbm,
             ex_hbm, dpart_hbm,
             sidx, nidx, didx, asg0, asg1, adg0, adg1, ae0, ae1, ex0, ex1,
             dacc, sa0, sa1, sb0, sb1, sc0, sc1, sw0, sw1, ss0, ss1):
    c = lax.axis_index("c")
    s = lax.axis_index("s")
    wid = c * NS + s
    asg = (asg0, asg1)
    adg = (adg0, adg1)
    ae = (ae0, ae1)
    ex = (ex0, ex1)
    sa = (sa0, sa1)
    sb = (sb0, sb1)
    se = (sc0, sc1)
    sw = (sw0, sw1)
    ss = (ss0, ss1)

    @pl.when(s == 0)
    def _():
        pltpu.sync_copy(zero_hbm, dacc)

    pltpu.sync_copy(srcv_hbm.at[pl.ds(wid * CPW, CPW)], sidx)
    pltpu.sync_copy(dstnv_hbm.at[pl.ds(wid * CPW, CPW)], nidx)
    pltpu.sync_copy(dstv_hbm.at[pl.ds(wid * CPW, CPW)], didx)
    plsc.subcore_barrier()

    def start(i, b):
        pltpu.async_copy(aa_hbm.at[sidx.at[i]], asg[b], sa[b])
        pltpu.async_copy(aa_hbm.at[nidx.at[i]], adg[b], sb[b])
        pltpu.async_copy(aedge_hbm.at[pl.ds((wid * CPW + i) * CH, CH)],
                         ae[b], se[b])

    def wait(b):
        pltpu.make_async_copy(aa_hbm.at[sidx.at[0]], asg[b], sa[b]).wait()
        pltpu.make_async_copy(aa_hbm.at[nidx.at[0]], adg[b], sb[b]).wait()
        pltpu.make_async_copy(aedge_hbm.at[pl.ds(0, CH)], ae[b], se[b]).wait()

    def wait_out(b):
        pltpu.make_async_copy(ex[b], ex_hbm.at[pl.ds(0, CH)], sw[b]).wait()
        pltpu.make_async_copy(ex[b], dacc.at[didx.at[0]], ss[b]).wait()

    def step(i, b, b2):
        wait(b)

        @pl.when(i + 1 < CPW)
        def _():
            start(i + 1, b2)

        @pl.when(i >= 2)
        def _():
            wait_out(b)

        @pl.loop(0, CH, step=4)
        def _(k):
            for j in range(4):
                a = asg[b][k + j, :] + adg[b][k + j, :] + ae[b][k + j, :]
                a = jnp.maximum(a, 0.2 * a)
                ex[b][k + j, :] = jnp.exp(a)

        pltpu.async_copy(ex[b], ex_hbm.at[pl.ds((wid * CPW + i) * CH, CH)],
                         sw[b])
        pltpu.async_copy(ex[b], dacc.at[didx.at[i]], ss[b], add=True)

    start(0, 0)

    @pl.loop(0, CPW)
    def _(i):
        even = (i % 2) == 0

        @pl.when(even)
        def _():
            step(i, 0, 1)

        @pl.when(jnp.logical_not(even))
        def _():
            step(i, 1, 0)

    wait_out(0)
    wait_out(1)
    plsc.subcore_barrier()
    _dump_acc(dacc, dpart_hbm.at[c], s)


@functools.partial(
    pl.kernel,
    mesh=_mesh,
    compiler_params=_sc_params,
    out_type=jax.ShapeDtypeStruct((NC, N, C), _f32),
    scratch_types=[
        pltpu.VMEM((CPW, CH), jnp.int32),
        pltpu.VMEM((CPW, CH), jnp.int32),
        pltpu.VMEM((CH // 2, H * C), _bf16),
        pltpu.VMEM((CH // 2, H * C), _bf16),
        pltpu.VMEM((CH, HP), _f32),
        pltpu.VMEM((CH, HP), _f32),
        pltpu.VMEM((CH, HP), _f32),
        pltpu.VMEM((CH, HP), _f32),
        pltpu.VMEM((CH, HP), _f32),
        pltpu.VMEM((CH, HP), _f32),
        pltpu.VMEM((CH, C), _f32),
        pltpu.VMEM((CH, C), _f32),
        pltpu.VMEM_SHARED((N, C), _f32),
        pltpu.SemaphoreType.DMA,
        pltpu.SemaphoreType.DMA,
        pltpu.SemaphoreType.DMA,
        pltpu.SemaphoreType.DMA,
        pltpu.SemaphoreType.DMA,
        pltpu.SemaphoreType.DMA,
        pltpu.SemaphoreType.DMA,
        pltpu.SemaphoreType.DMA,
        pltpu.SemaphoreType.DMA,
        pltpu.SemaphoreType.DMA,
    ],
)
def _sc_msg(srcv_hbm, dstv_hbm, xp_hbm, ex_hbm, d0_hbm, d1_hbm, zero_hbm,
            outp_hbm,
            sidx, didx, xp0, xp1, exb0, exb1, d00, d01, d10, d11,
            msg0, msg1, oacc,
            sx0, sx1, sE0, sE1, s00, s01, s10, s11, sm0, sm1):
    c = lax.axis_index("c")
    s = lax.axis_index("s")
    wid = c * NS + s
    HB = CH // 2
    xp = (xp0, xp1)
    exb = (exb0, exb1)
    d0 = (d00, d01)
    d1 = (d10, d11)
    msg = (msg0, msg1)
    sx = (sx0, sx1)
    sE = (sE0, sE1)
    s0 = (s00, s01)
    s1 = (s10, s11)
    sm = (sm0, sm1)

    @pl.when(s == 0)
    def _():
        pltpu.sync_copy(zero_hbm, oacc)

    pltpu.sync_copy(srcv_hbm.at[pl.ds(wid * CPW, CPW)], sidx)
    pltpu.sync_copy(dstv_hbm.at[pl.ds(wid * CPW, CPW)], didx)
    plsc.subcore_barrier()

    def start_small(i, b):
        pltpu.async_copy(d0_hbm.at[didx.at[i]], d0[b], s0[b])
        pltpu.async_copy(d1_hbm.at[didx.at[i]], d1[b], s1[b])
        pltpu.async_copy(ex_hbm.at[pl.ds((wid * CPW + i) * CH, CH)],
                         exb[b], sE[b])

    def wait_small(b):
        pltpu.make_async_copy(d0_hbm.at[didx.at[0]], d0[b], s0[b]).wait()
        pltpu.make_async_copy(d1_hbm.at[didx.at[0]], d1[b], s1[b]).wait()
        pltpu.make_async_copy(ex_hbm.at[pl.ds(0, CH)], exb[b], sE[b]).wait()

    def start_xp(i, hf, xb):
        pltpu.async_copy(xp_hbm.at[sidx.at[i, pl.ds(hf * HB, HB)]],
                         xp[xb], sx[xb])

    def wait_xp(xb):
        pltpu.make_async_copy(xp_hbm.at[sidx.at[0, pl.ds(0, HB)]],
                              xp[xb], sx[xb]).wait()

    def compute_half(b, hf, xb):
        base = hf * HB

        @pl.loop(0, HB, step=2)
        def _(k0):
            for j in range(2):
                k = base + k0 + j
                den = d0[b][k, :] + d1[b][k, :] + 1e-16
                wrow = exb[b][k, :] / den * (1.0 / H)
                accs = [jnp.zeros((16,), _f32) for _ in range(4)]
                for hh in range(H):
                    wk = wrow[hh]
                    for half in range(2):
                        v = xp[xb][k0 + j, pl.ds(hh * C + half * 32, 32)]
                        lo, hi = plsc.unpack(
                            v, format=plsc.PackFormat.INTERLEAVED)
                        accs[2 * half] = accs[2 * half] + wk * lo
                        accs[2 * half + 1] = accs[2 * half + 1] + wk * hi
                for q in range(4):
                    msg[b][k, pl.ds(q * 16, 16)] = accs[q]

    def wait_out(b):
        pltpu.make_async_copy(msg[b], oacc.at[didx.at[0]], sm[b]).wait()

    def step(i, b, b2):
        wait_small(b)

        @pl.when(i + 1 < CPW)
        def _():
            start_small(i + 1, b2)

        @pl.when(i >= 2)
        def _():
            wait_out(b)

        wait_xp(0)
        start_xp(i, 1, 1)
        compute_half(b, 0, 0)
        wait_xp(1)

        @pl.when(i + 1 < CPW)
        def _():
            start_xp(i + 1, 0, 0)

        compute_half(b, 1, 1)
        pltpu.async_copy(msg[b], oacc.at[didx.at[i]], sm[b], add=True)

    start_small(0, 0)
    start_xp(0, 0, 0)

    @pl.loop(0, CPW)
    def _(i):
        even = (i % 2) == 0

        @pl.when(even)
        def _():
            step(i, 0, 1)

        @pl.when(jnp.logical_not(even))
        def _():
            step(i, 1, 0)

    wait_out(0)
    wait_out(1)
    plsc.subcore_barrier()
    _dump_acc(oacc, outp_hbm.at[c], s)


def kernel(x, edge_index, edge_attr, batch, W_embed, b_embed, convW, edgeW,
           att_src, att_dst, att_edge, conv_bias, bn_gamma, bn_beta,
           linW, linb):
    padi = jnp.arange(EPAD - E, dtype=jnp.int32) % N
    src1 = jnp.concatenate([edge_index[0].astype(jnp.int32), padi])
    dst1 = jnp.concatenate([edge_index[1].astype(jnp.int32), padi])
    srcv = src1.reshape(EPAD // CH, CH)
    dstv = dst1.reshape(EPAD // CH, CH)
    dstnv = dstv + N

    cw = convW.reshape(L, C, H, C)
    ws = jnp.einsum("lchd,lhd->lch", cw, att_src)
    wd = jnp.einsum("lchd,lhd->lch", cw, att_dst)
    ew = edgeW.reshape(L, D_EDGE, H, C)
    we = jnp.einsum("lehd,lhd->leh", ew, att_edge)
    pad = ((0, 0), (0, 0), (0, HP - H))
    ws = jnp.pad(ws, pad).astype(_bf16)
    wd = jnp.pad(wd, pad).astype(_bf16)
    we = jnp.pad(we, pad)
    cwp = convW[:, :, _XPERM].astype(_bf16)
    web = W_embed.astype(_bf16)

    zeros16 = jnp.zeros((N, HP), _f32)
    zeros64 = jnp.zeros((N, C), _f32)
    ea_pad = jnp.concatenate(
        [edge_attr, jnp.zeros((EPAD - E, D_EDGE), _f32)])

    aedge0, aedge1 = _edge_call(ea_pad, we[0], we[1])

    h0, xp0, aa0 = _node0_call(x, web, b_embed.reshape(1, C),
                               cwp[0], ws[0], wd[0])

    ex0, dp0 = _sc_attn(srcv, dstnv, dstv, aa0, aedge0, zeros16)
    op0 = _sc_msg(srcv, dstv, xp0, ex0, dp0[0], dp0[1], zeros64)

    h1, xp1, aa1 = _post_call(op0[0], op0[1],
                              conv_bias[0].reshape(1, C),
                              bn_gamma[0].reshape(1, C),
                              bn_beta[0].reshape(1, C),
                              h0, cwp[1], ws[1], wd[1])

    ex1, dp1 = _sc_attn(srcv, dstnv, dstv, aa1, aedge1, zeros16)
    op1 = _sc_msg(srcv, dstv, xp1, ex1, dp1[0], dp1[1], zeros64)

    pooled = _final_call(op1[0], op1[1],
                         conv_bias[1].reshape(1, C),
                         bn_gamma[1].reshape(1, C),
                         bn_beta[1].reshape(1, C),
                         h1, linW, linb.reshape(1, OUT),
                         batch.reshape(1, N).astype(jnp.int32))
    return pooled


# speedup vs baseline: 1.4179x; 1.0092x over previous
import functools

import jax
import jax.numpy as jnp
import numpy as np
from jax import lax
from jax.experimental import pallas as pl
from jax.experimental.pallas import tpu as pltpu
from jax.experimental.pallas import tpu_sc as plsc

N = 10000
E = 160000
D_IN = 128
D_EDGE = 16
H = 10
HP = 16
C = 64
L = 2
G = 64
OUT = 128

NC = 2
NS = 16
NW = NC * NS
CH = 128
EPAD = 163840
CPW = EPAD // (NW * CH)
ROWS_PER_TILE = 624
ROWS_LAST = N - (NS - 1) * ROWS_PER_TILE

_f32 = jnp.float32
_bf16 = jnp.bfloat16
_XPERM = np.concatenate([
    32 * g + np.ravel(np.stack([np.arange(16), 16 + np.arange(16)], axis=1))
    for g in range(H * C // 32)
])
_mesh = plsc.VectorSubcoreMesh(core_axis_name="c", subcore_axis_name="s")
_sc_params = pltpu.CompilerParams(use_tc_tiling_on_sc=False,
                                  needs_layout_passes=False)
_tc_params = pltpu.CompilerParams(vmem_limit_bytes=64 * 1024 * 1024)


def _node0_body(x_ref, we_ref, be_ref, cw_ref, ws_ref, wd_ref,
                h_ref, xp_ref, aa_ref):
    h = jnp.dot(x_ref[...], we_ref[...], preferred_element_type=_f32)
    h = h + be_ref[...]
    h_ref[...] = h
    hb = h.astype(_bf16)
    xp_ref[...] = jnp.dot(hb, cw_ref[...],
                          preferred_element_type=_f32).astype(_bf16)
    aa_ref[0:N, :] = jnp.dot(hb, ws_ref[...], preferred_element_type=_f32)
    aa_ref[N:2 * N, :] = jnp.dot(hb, wd_ref[...], preferred_element_type=_f32)


def _edge_body(ea_ref, w0_ref, w1_ref, o0_ref, o1_ref):
    i = pl.program_id(0)
    ea = ea_ref[...]
    rows = i * _EB + lax.broadcasted_iota(jnp.int32, (_EB, HP), 0)
    pad = rows >= E
    o0 = jnp.dot(ea, w0_ref[...], preferred_element_type=_f32)
    o1 = jnp.dot(ea, w1_ref[...], preferred_element_type=_f32)
    o0_ref[...] = jnp.where(pad, -1e30, o0)
    o1_ref[...] = jnp.where(pad, -1e30, o1)


def _bn_gelu_res(o0, o1, bias, gamma, beta, hprev):
    out = o0 + o1 + bias
    mu = jnp.mean(out, axis=0, keepdims=True)
    var = jnp.mean((out - mu) ** 2, axis=0, keepdims=True)
    xn = (out - mu) / jnp.sqrt(var + 1e-5) * gamma + beta
    g = 0.5 * xn * (1.0 + lax.erf(xn / jnp.sqrt(2.0).astype(_f32)))
    return hprev + g


def _post_body(o0_ref, o1_ref, bias_ref, gam_ref, bet_ref, hp_ref,
               cw_ref, ws_ref, wd_ref,
               h_ref, xp_ref, aa_ref):
    h = _bn_gelu_res(o0_ref[...], o1_ref[...], bias_ref[...], gam_ref[...],
                     bet_ref[...], hp_ref[...])
    h_ref[...] = h
    hb = h.astype(_bf16)
    xp_ref[...] = jnp.dot(hb, cw_ref[...],
                          preferred_element_type=_f32).astype(_bf16)
    aa_ref[0:N, :] = jnp.dot(hb, ws_ref[...], preferred_element_type=_f32)
    aa_ref[N:2 * N, :] = jnp.dot(hb, wd_ref[...], preferred_element_type=_f32)


def _final_body(o0_ref, o1_ref, bias_ref, gam_ref, bet_ref, hp_ref,
                lw_ref, lb_ref, batch_ref, out_ref):
    h = _bn_gelu_res(o0_ref[...], o1_ref[...], bias_ref[...], gam_ref[...],
                     bet_ref[...], hp_ref[...])
    y = jnp.dot(h, lw_ref[...], preferred_element_type=_f32) + lb_ref[...]
    gids = lax.broadcasted_iota(jnp.int32, (G, N), 0)
    oh = (gids == batch_ref[...]).astype(_f32)
    ssum = jnp.dot(oh, y, preferred_element_type=_f32)
    cnt = jnp.sum(oh, axis=1, keepdims=True)
    out_ref[...] = ssum / jnp.maximum(cnt, 1.0)


_node0_call = pl.pallas_call(
    _node0_body,
    compiler_params=_tc_params,
    out_shape=[
        jax.ShapeDtypeStruct((N, C), _f32),
        jax.ShapeDtypeStruct((N, H * C), _bf16),
        jax.ShapeDtypeStruct((2 * N, HP), _f32),
    ],
)

_EB = 8192
_edge_call = pl.pallas_call(
    _edge_body,
    grid=(EPAD // _EB,),
    in_specs=[
        pl.BlockSpec((_EB, D_EDGE), lambda i: (i, 0)),
        pl.BlockSpec((D_EDGE, HP), lambda i: (0, 0)),
        pl.BlockSpec((D_EDGE, HP), lambda i: (0, 0)),
    ],
    out_specs=[
        pl.BlockSpec((_EB, HP), lambda i: (i, 0)),
        pl.BlockSpec((_EB, HP), lambda i: (i, 0)),
    ],
    out_shape=[
        jax.ShapeDtypeStruct((EPAD, HP), _f32),
        jax.ShapeDtypeStruct((EPAD, HP), _f32),
    ],
)

_post_call = pl.pallas_call(
    _post_body,
    compiler_params=_tc_params,
    out_shape=[
        jax.ShapeDtypeStruct((N, C), _f32),
        jax.ShapeDtypeStruct((N, H * C), _bf16),
        jax.ShapeDtypeStruct((2 * N, HP), _f32),
    ],
)

_final_call = pl.pallas_call(
    _final_body,
    compiler_params=_tc_params,
    out_shape=jax.ShapeDtypeStruct((G, OUT), _f32),
)


def _dump_acc(acc, out_hbm, s):
    @pl.when(s < NS - 1)
    def _():
        pltpu.sync_copy(acc.at[pl.ds(s * ROWS_PER_TILE, ROWS_PER_TILE)],
                        out_hbm.at[pl.ds(s * ROWS_PER_TILE, ROWS_PER_TILE)])

    @pl.when(s == NS - 1)
    def _():
        pltpu.sync_copy(acc.at[pl.ds((NS - 1) * ROWS_PER_TILE, ROWS_LAST)],
                        out_hbm.at[pl.ds((NS - 1) * ROWS_PER_TILE, ROWS_LAST)])


@functools.partial(
    pl.kernel,
    mesh=_mesh,
    compiler_params=_sc_params,
    out_type=[
        jax.ShapeDtypeStruct((EPAD, HP), _f32),
        jax.ShapeDtypeStruct((NC, N, HP), _f32),
    ],
    scratch_types=[
        pltpu.VMEM((CPW, CH), jnp.int32),
        pltpu.VMEM((CPW, CH), jnp.int32),
        pltpu.VMEM((CPW, CH), jnp.int32),
        pltpu.VMEM((CH, HP), _f32),
        pltpu.VMEM((CH, HP), _f32),
        pltpu.VMEM((CH, HP), _f32),
        pltpu.VMEM((CH, HP), _f32),
        pltpu.VMEM((CH, HP), _f32),
        pltpu.VMEM((CH, HP), _f32),
        pltpu.VMEM((CH, HP), _f32),
        pltpu.VMEM((CH, HP), _f32),
        pltpu.VMEM_SHARED((N, HP), _f32),
        pltpu.SemaphoreType.DMA,
        pltpu.SemaphoreType.DMA,
        pltpu.SemaphoreType.DMA,
        pltpu.SemaphoreType.DMA,
        pltpu.SemaphoreType.DMA,
        pltpu.SemaphoreType.DMA,
        pltpu.SemaphoreType.DMA,
        pltpu.SemaphoreType.DMA,
        pltpu.SemaphoreType.DMA,
        pltpu.SemaphoreType.DMA,
    ],
)
def _sc_attn(srcv_hbm, dstnv_hbm, dstv_hbm, aa_hbm, aedge_hbm, zero_hbm,
             ex_hbm, dpart_hbm,
             sidx, nidx, didx, asg0, asg1, adg0, adg1, ae0, ae1, ex0, ex1,
             dacc, sa0, sa1, sb0, sb1, sc0, sc1, sw0, sw1, ss0, ss1):
    c = lax.axis_index("c")
    s = lax.axis_index("s")
    wid = c * NS + s
    asg = (asg0, asg1)
    adg = (adg0, adg1)
    ae = (ae0, ae1)
    ex = (ex0, ex1)
    sa = (sa0, sa1)
    sb = (sb0, sb1)
    se = (sc0, sc1)
    sw = (sw0, sw1)
    ss = (ss0, ss1)

    @pl.when(s == 0)
    def _():
        pltpu.sync_copy(zero_hbm, dacc)

    pltpu.sync_copy(srcv_hbm.at[pl.ds(wid * CPW, CPW)], sidx)
    pltpu.sync_copy(dstnv_hbm.at[pl.ds(wid * CPW, CPW)], nidx)
    pltpu.sync_copy(dstv_hbm.at[pl.ds(wid * CPW, CPW)], didx)
    plsc.subcore_barrier()

    def start(i, b):
        pltpu.async_copy(aa_hbm.at[sidx.at[i]], asg[b], sa[b])
        pltpu.async_copy(aa_hbm.at[nidx.at[i]], adg[b], sb[b])
        pltpu.async_copy(aedge_hbm.at[pl.ds((wid * CPW + i) * CH, CH)],
                         ae[b], se[b])

    def wait(b):
        pltpu.make_async_copy(aa_hbm.at[sidx.at[0]], asg[b], sa[b]).wait()
        pltpu.make_async_copy(aa_hbm.at[nidx.at[0]], adg[b], sb[b]).wait()
        pltpu.make_async_copy(aedge_hbm.at[pl.ds(0, CH)], ae[b], se[b]).wait()

    def wait_out(b):
        pltpu.make_async_copy(ex[b], ex_hbm.at[pl.ds(0, CH)], sw[b]).wait()
        pltpu.make_async_copy(ex[b], dacc.at[didx.at[0]], ss[b]).wait()

    def step(i, b, b2):
        wait(b)

        @pl.when(i + 1 < CPW)
        def _():
            start(i + 1, b2)

        @pl.when(i >= 2)
        def _():
            wait_out(b)

        @pl.loop(0, CH, step=4)
        def _(k):
            for j in range(4):
                a = asg[b][k + j, :] + adg[b][k + j, :] + ae[b][k + j, :]
                a = jnp.maximum(a, 0.2 * a)
                ex[b][k + j, :] = jnp.exp(a)

        pltpu.async_copy(ex[b], ex_hbm.at[pl.ds((wid * CPW + i) * CH, CH)],
                         sw[b])
        pltpu.async_copy(ex[b], dacc.at[didx.at[i]], ss[b], add=True)

    start(0, 0)

    @pl.loop(0, CPW)
    def _(i):
        even = (i % 2) == 0

        @pl.when(even)
        def _():
            step(i, 0, 1)

        @pl.when(jnp.logical_not(even))
        def _():
            step(i, 1, 0)

    wait_out(0)
    wait_out(1)
    plsc.subcore_barrier()
    _dump_acc(dacc, dpart_hbm.at[c], s)


@functools.partial(
    pl.kernel,
    mesh=_mesh,
    compiler_params=_sc_params,
    out_type=jax.ShapeDtypeStruct((NC, N, C), _f32),
    scratch_types=[
        pltpu.VMEM((CPW, CH), jnp.int32),
        pltpu.VMEM((CPW, CH), jnp.int32),
        pltpu.VMEM((CH // 2, H * C), _bf16),
        pltpu.VMEM((CH // 2, H * C), _bf16),
        pltpu.VMEM((CH, HP), _f32),
        pltpu.VMEM((CH, HP), _f32),
        pltpu.VMEM((CH, HP), _f32),
        pltpu.VMEM((CH, HP), _f32),
        pltpu.VMEM((CH, HP), _f32),
        pltpu.VMEM((CH, HP), _f32),
        pltpu.VMEM((CH, C), _f32),
        pltpu.VMEM((CH, C), _f32),
        pltpu.VMEM_SHARED((N, C), _f32),
        pltpu.SemaphoreType.DMA,
        pltpu.SemaphoreType.DMA,
        pltpu.SemaphoreType.DMA,
        pltpu.SemaphoreType.DMA,
        pltpu.SemaphoreType.DMA,
        pltpu.SemaphoreType.DMA,
        pltpu.SemaphoreType.DMA,
        pltpu.SemaphoreType.DMA,
        pltpu.SemaphoreType.DMA,
        pltpu.SemaphoreType.DMA,
    ],
)
def _sc_msg(srcv_hbm, dstv_hbm, xp_hbm, ex_hbm, d0_hbm, d1_hbm, zero_hbm,
            outp_hbm,
            sidx, didx, xp0, xp1, exb0, exb1, d00, d01, d10, d11,
            msg0, msg1, oacc,
            sx0, sx1, sE0, sE1, s00, s01, s10, s11, sm0, sm1):
    c = lax.axis_index("c")
    s = lax.axis_index("s")
    wid = c * NS + s
    HB = CH // 2
    xp = (xp0, xp1)
    exb = (exb0, exb1)
    d0 = (d00, d01)
    d1 = (d10, d11)
    msg = (msg0, msg1)
    sx = (sx0, sx1)
    sE = (sE0, sE1)
    s0 = (s00, s01)
    s1 = (s10, s11)
    sm = (sm0, sm1)

    @pl.when(s == 0)
    def _():
        pltpu.sync_copy(zero_hbm, oacc)

    pltpu.sync_copy(srcv_hbm.at[pl.ds(wid * CPW, CPW)], sidx)
    pltpu.sync_copy(dstv_hbm.at[pl.ds(wid * CPW, CPW)], didx)
    plsc.subcore_barrier()

    def start_small(i, b):
        pltpu.async_copy(d0_hbm.at[didx.at[i]], d0[b], s0[b])
        pltpu.async_copy(d1_hbm.at[didx.at[i]], d1[b], s1[b])
        pltpu.async_copy(ex_hbm.at[pl.ds((wid * CPW + i) * CH, CH)],
                         exb[b], sE[b])

    def wait_small(b):
        pltpu.make_async_copy(d0_hbm.at[didx.at[0]], d0[b], s0[b]).wait()
        pltpu.make_async_copy(d1_hbm.at[didx.at[0]], d1[b], s1[b]).wait()
        pltpu.make_async_copy(ex_hbm.at[pl.ds(0, CH)], exb[b], sE[b]).wait()

    def start_xp(i, hf, xb):
        pltpu.async_copy(xp_hbm.at[sidx.at[i, pl.ds(hf * HB, HB)]],
                         xp[xb], sx[xb])

    def wait_xp(xb):
        pltpu.make_async_copy(xp_hbm.at[sidx.at[0, pl.ds(0, HB)]],
                              xp[xb], sx[xb]).wait()

    def compute_half(b, hf, xb):
        base = hf * HB

        @pl.loop(0, HB, step=2)
        def _(k0):
            for j in range(2):
                k = base + k0 + j
                den = d0[b][k, :] + d1[b][k, :] + 1e-16
                wrow = exb[b][k, :] / den * (1.0 / H)
                accs = [jnp.zeros((16,), _f32) for _ in range(4)]
                for hh in range(H):
                    wk = wrow[hh]
                    for half in range(2):
                        v = xp[xb][k0 + j, pl.ds(hh * C + half * 32, 32)]
                        lo, hi = plsc.unpack(
                            v, format=plsc.PackFormat.INTERLEAVED)
                        accs[2 * half] = accs[2 * half] + wk * lo
                        accs[2 * half + 1] = accs[2 * half + 1] + wk * hi
                for q in range(4):
                    msg[b][k, pl.ds(q * 16, 16)] = accs[q]

    def wait_out(b):
        pltpu.make_async_copy(msg[b], oacc.at[didx.at[0]], sm[b]).wait()

    def step(i, b, b2):
        wait_small(b)

        @pl.when(i + 1 < CPW)
        def _():
            start_small(i + 1, b2)

        @pl.when(i >= 2)
        def _():
            wait_out(b)

        wait_xp(0)
        start_xp(i, 1, 1)
        compute_half(b, 0, 0)
        wait_xp(1)

        @pl.when(i + 1 < CPW)
        def _():
            start_xp(i + 1, 0, 0)

        compute_half(b, 1, 1)
        pltpu.async_copy(msg[b], oacc.at[didx.at[i]], sm[b], add=True)

    start_small(0, 0)
    start_xp(0, 0, 0)

    @pl.loop(0, CPW)
    def _(i):
        even = (i % 2) == 0

        @pl.when(even)
        def _():
            step(i, 0, 1)

        @pl.when(jnp.logical_not(even))
        def _():
            step(i, 1, 0)

    wait_out(0)
    wait_out(1)
    plsc.subcore_barrier()
    _dump_acc(oacc, outp_hbm.at[c], s)


def kernel(x, edge_index, edge_attr, batch, W_embed, b_embed, convW, edgeW,
           att_src, att_dst, att_edge, conv_bias, bn_gamma, bn_beta,
           linW, linb):
    padi = jnp.arange(EPAD - E, dtype=jnp.int32) % N
    src1 = jnp.concatenate([edge_index[0].astype(jnp.int32), padi])
    dst1 = jnp.concatenate([edge_index[1].astype(jnp.int32), padi])
    srcv = src1.reshape(EPAD // CH, CH)
    dstv = dst1.reshape(EPAD // CH, CH)
    dstnv = dstv + N

    cw = convW.reshape(L, C, H, C)
    ws = jnp.einsum("lchd,lhd->lch", cw, att_src)
    wd = jnp.einsum("lchd,lhd->lch", cw, att_dst)
    ew = edgeW.reshape(L, D_EDGE, H, C)
    we = jnp.einsum("lehd,lhd->leh", ew, att_edge)
    pad = ((0, 0), (0, 0), (0, HP - H))
    ws = jnp.pad(ws, pad).astype(_bf16)
    wd = jnp.pad(wd, pad).astype(_bf16)
    we = jnp.pad(we, pad)
    cwp = convW[:, :, _XPERM].astype(_bf16)
    web = W_embed.astype(_bf16)

    zeros16 = jnp.zeros((N, HP), _f32)
    zeros64 = jnp.zeros((N, C), _f32)

    aedge0, aedge1 = _edge_call(edge_attr, we[0], we[1])

    h0, xp0, aa0 = _node0_call(x.astype(_bf16), web, b_embed.reshape(1, C),
                               cwp[0], ws[0], wd[0])

    ex0, dp0 = _sc_attn(srcv, dstnv, dstv, aa0, aedge0, zeros16)
    op0 = _sc_msg(srcv, dstv, xp0, ex0, dp0[0], dp0[1], zeros64)

    h1, xp1, aa1 = _post_call(op0[0], op0[1],
                              conv_bias[0].reshape(1, C),
                              bn_gamma[0].reshape(1, C),
                              bn_beta[0].reshape(1, C),
                              h0, cwp[1], ws[1], wd[1])

    ex1, dp1 = _sc_attn(srcv, dstnv, dstv, aa1, aedge1, zeros16)
    op1 = _sc_msg(srcv, dstv, xp1, ex1, dp1[0], dp1[1], zeros64)

    pooled = _final_call(op1[0], op1[1],
                         conv_bias[1].reshape(1, C),
                         bn_gamma[1].reshape(1, C),
                         bn_beta[1].reshape(1, C),
                         h1, linW, linb.reshape(1, OUT),
                         batch.reshape(1, N).astype(jnp.int32))
    return pooled
